# baseline (speedup 1.0000x reference)
import functools

import jax
import jax.numpy as jnp
from jax import lax
from jax.experimental import pallas as pl
from jax.experimental.pallas import tpu as pltpu
from jax.experimental.pallas import tpu_sc as plsc

_N_EDGES = 320000
_N_NODES = 10000
_D = 128
_DP = _D // 2
_NC = 2
_NS = 16
_NW = _NC * _NS
_EW = _N_EDGES // _NW
_C = 80
_NCHUNK = _EW // _C
_G = _C // 16


def _body(x_hbm, src_hbm, dst_hbm, out_hbm,
          idx_u, idx_v, xs, ru0, ru1, rv0, rv1, out_v,
          su0, su1, sv0, sv1):
    cid = lax.axis_index("c")
    sid = lax.axis_index("s")
    wid = sid * _NC + cid

    pltpu.sync_copy(src_hbm.at[wid], idx_u)
    pltpu.sync_copy(dst_hbm.at[wid], idx_v)

    stripe = _N_NODES // _NS
    pltpu.sync_copy(x_hbm.at[pl.ds(sid * stripe, stripe)],
                    xs.at[pl.ds(sid * stripe, stripe)])
    plsc.subcore_barrier()

    rus = (ru0, ru1)
    rvs = (rv0, rv1)
    sus = (su0, su1)
    svs = (sv0, sv1)

    def fire(c, b):
        pltpu.async_copy(x_hbm.at[idx_u.at[c]], rus[b], sus[b])
        pltpu.async_copy(xs.at[idx_v.at[c]], rvs[b], svs[b])

    def wait(b):
        pltpu.make_async_copy(x_hbm.at[idx_u.at[0]], rus[b], sus[b]).wait()
        pltpu.make_async_copy(xs.at[idx_v.at[0]], rvs[b], svs[b]).wait()

    iota = lax.iota(jnp.int32, 16)

    def comp(c, b):
        rows_u, rows_v = rus[b], rvs[b]
        for g in range(_G):
            rows = g * 16 + iota

            def dstep(t, accs):
                a0, a1, a2, a3 = accs
                cols0 = iota + t * 4
                accs_new = [a0, a1, a2, a3]
                for k in range(4):
                    cols = (cols0 + k) & (_DP - 1)
                    u = plsc.bitcast(plsc.load_gather(rows_u, [rows, cols]),
                                     jnp.bfloat16)
                    v = plsc.bitcast(plsc.load_gather(rows_v, [rows, cols]),
                                     jnp.bfloat16)
                    lo, hi = plsc.unpack(u * v, format=plsc.PackFormat.INTERLEAVED)
                    accs_new[(2 * k) % 4] = accs_new[(2 * k) % 4] + lo
                    accs_new[(2 * k + 1) % 4] = accs_new[(2 * k + 1) % 4] + hi
                return tuple(accs_new)

            z = jnp.zeros((16,), jnp.float32)
            a0, a1, a2, a3 = lax.fori_loop(0, _DP // 4, dstep, (z, z, z, z))
            out_v[pl.ds(c * _C + g * 16, 16)] = (a0 + a1) + (a2 + a3)

    fire(0, 0)

    def loop_body(t, _):
        for b in range(2):
            c = 2 * t + b
            wait(b)
            fire(c + 1, 1 - b)
            comp(c, b)
        return 0

    lax.fori_loop(0, (_NCHUNK - 1) // 2, loop_body, 0)
    wait(0)
    comp(_NCHUNK - 1, 0)

    pltpu.sync_copy(out_v, out_hbm.at[pl.ds(wid * _EW, _EW)])


@functools.partial(jax.jit, static_argnums=())
def kernel(x, edge_index):
    src = edge_index[0].astype(jnp.int32).reshape(_NW, _NCHUNK, _C)
    dst = edge_index[1].astype(jnp.int32).reshape(_NW, _NCHUNK, _C)
    x_packed = jax.lax.bitcast_convert_type(
        x.astype(jnp.bfloat16).reshape(_N_NODES, _DP, 2), jnp.int32)
    mesh = plsc.VectorSubcoreMesh(core_axis_name="c", subcore_axis_name="s")
    call = pl.kernel(
        _body,
        out_type=jax.ShapeDtypeStruct((_N_EDGES,), jnp.float32),
        mesh=mesh,
        scratch_types=[
            pltpu.VMEM((_NCHUNK, _C), jnp.int32),
            pltpu.VMEM((_NCHUNK, _C), jnp.int32),
            pltpu.VMEM_SHARED((_N_NODES, _DP), jnp.int32),
            pltpu.VMEM((_C, _DP), jnp.int32),
            pltpu.VMEM((_C, _DP), jnp.int32),
            pltpu.VMEM((_C, _DP), jnp.int32),
            pltpu.VMEM((_C, _DP), jnp.int32),
            pltpu.VMEM((_EW,), jnp.float32),
            pltpu.SemaphoreType.DMA,
            pltpu.SemaphoreType.DMA,
            pltpu.SemaphoreType.DMA,
            pltpu.SemaphoreType.DMA,
        ],
        compiler_params=pltpu.CompilerParams(
            needs_layout_passes=False, use_tc_tiling_on_sc=False),
    )
    score = call(x_packed, src, dst)
    return score.reshape(_N_EDGES, 1)


# speedup vs baseline: 1.0687x; 1.0687x over previous
import functools

import jax
import jax.numpy as jnp
from jax import lax
from jax.experimental import pallas as pl
from jax.experimental.pallas import tpu as pltpu
from jax.experimental.pallas import tpu_sc as plsc

_N_EDGES = 320000
_N_NODES = 10000
_D = 128
_DP = _D // 2
_NC = 2
_NS = 16
_NW = _NC * _NS
_EW = _N_EDGES // _NW
_C = 200
_NCHUNK = _EW // _C
_G = (_C + 15) // 16


def _body(x_hbm, idx_hbm, out_hbm,
          idx_all, xs, r0, r1, o0, o1,
          sr0, sr1, so0, so1):
    cid = lax.axis_index("c")
    sid = lax.axis_index("s")
    wid = sid * _NC + cid

    pltpu.sync_copy(idx_hbm.at[wid], idx_all)

    stripe = _N_NODES // _NS
    pltpu.sync_copy(x_hbm.at[pl.ds(sid * stripe, stripe)],
                    xs.at[pl.ds(sid * stripe, stripe)])
    plsc.subcore_barrier()

    rbufs = (r0, r1)
    obufs = (o0, o1)
    rsems = (sr0, sr1)
    osems = (so0, so1)

    def fire(c, b):
        pltpu.async_copy(xs.at[idx_all.at[c]], rbufs[b], rsems[b])

    def wait(b):
        pltpu.make_async_copy(xs.at[idx_all.at[0]], rbufs[b], rsems[b]).wait()

    iota = lax.iota(jnp.int32, 16)

    def comp(c, b):
        rows_buf = rbufs[b]
        out_b = obufs[b]

        @pl.when(c >= 2)
        def _():
            pltpu.make_async_copy(out_b.at[pl.ds(0, _C)],
                                  out_hbm.at[pl.ds(0, _C)],
                                  osems[b]).wait()

        for g in range(_G):
            rows_u = g * 16 + iota
            rows_v = rows_u + _C

            def dstep(t, accs):
                a0, a1, a2, a3 = accs
                cols0 = iota + t * 4
                accs_new = [a0, a1, a2, a3]
                for k in range(4):
                    cols = (cols0 + k) & (_DP - 1)
                    u = plsc.bitcast(plsc.load_gather(rows_buf, [rows_u, cols]),
                                     jnp.bfloat16)
                    v = plsc.bitcast(plsc.load_gather(rows_buf, [rows_v, cols]),
                                     jnp.bfloat16)
                    lo, hi = plsc.unpack(u * v, format=plsc.PackFormat.INTERLEAVED)
                    accs_new[(2 * k) % 4] = accs_new[(2 * k) % 4] + lo
                    accs_new[(2 * k + 1) % 4] = accs_new[(2 * k + 1) % 4] + hi
                return tuple(accs_new)

            z = jnp.zeros((16,), jnp.float32)
            a0, a1, a2, a3 = lax.fori_loop(0, _DP // 4, dstep, (z, z, z, z))
            out_b[pl.ds(g * 16, 16)] = (a0 + a1) + (a2 + a3)
        pltpu.async_copy(out_b.at[pl.ds(0, _C)],
                         out_hbm.at[pl.ds(wid * _EW + c * _C, _C)],
                         osems[b])

    fire(0, 0)

    def loop_body(t, _):
        for b in range(2):
            c = 2 * t + b
            wait(b)
            fire(c + 1, 1 - b)
            comp(c, b)
        return 0

    lax.fori_loop(0, (_NCHUNK - 1) // 2, loop_body, 0)
    wait(0)
    comp(_NCHUNK - 1, 0)
    pltpu.make_async_copy(o1.at[pl.ds(0, _C)], out_hbm.at[pl.ds(0, _C)],
                          so1).wait()
    pltpu.make_async_copy(o0.at[pl.ds(0, _C)], out_hbm.at[pl.ds(0, _C)],
                          so0).wait()


@functools.partial(jax.jit, static_argnums=())
def kernel(x, edge_index):
    src = edge_index[0].astype(jnp.int32).reshape(_NW, _NCHUNK, _C)
    dst = edge_index[1].astype(jnp.int32).reshape(_NW, _NCHUNK, _C)
    idx_cat = jnp.concatenate([src, dst], axis=2)
    x_packed = jax.lax.bitcast_convert_type(
        x.astype(jnp.bfloat16).reshape(_N_NODES, _DP, 2), jnp.int32)
    mesh = plsc.VectorSubcoreMesh(core_axis_name="c", subcore_axis_name="s")
    call = pl.kernel(
        _body,
        out_type=jax.ShapeDtypeStruct((_N_EDGES,), jnp.float32),
        mesh=mesh,
        scratch_types=[
            pltpu.VMEM((_NCHUNK, 2 * _C), jnp.int32),
            pltpu.VMEM_SHARED((_N_NODES, _DP), jnp.int32),
            pltpu.VMEM((2 * _C, _DP), jnp.int32),
            pltpu.VMEM((2 * _C, _DP), jnp.int32),
            pltpu.VMEM((16 * _G,), jnp.float32),
            pltpu.VMEM((16 * _G,), jnp.float32),
            pltpu.SemaphoreType.DMA,
            pltpu.SemaphoreType.DMA,
            pltpu.SemaphoreType.DMA,
            pltpu.SemaphoreType.DMA,
        ],
        compiler_params=pltpu.CompilerParams(
            needs_layout_passes=False, use_tc_tiling_on_sc=False),
    )
    score = call(x_packed, idx_cat)
    return score.reshape(_N_EDGES, 1)


# speedup vs baseline: 1.1486x; 1.0748x over previous
import functools

import jax
import jax.numpy as jnp
from jax import lax
from jax.experimental import pallas as pl
from jax.experimental.pallas import tpu as pltpu
from jax.experimental.pallas import tpu_sc as plsc

_N_EDGES = 320000
_N_NODES = 10000
_D = 128
_DP = _D // 2
_NC = 2
_NS = 16
_NW = _NC * _NS
_EW = _N_EDGES // _NW
_C = 80
_NCHUNK = _EW // _C
_G = _C // 16


def _body(x_hbm, idx_hbm, out_hbm,
          idx2, xs, ru0, ru1, rv0, rv1, out_v,
          su0, su1, sv0, sv1):
    cid = lax.axis_index("c")
    sid = lax.axis_index("s")
    wid = sid * _NC + cid

    stripe = _N_NODES // _NS
    cp_i = pltpu.async_copy(idx_hbm.at[wid], idx2, su0)
    cp_x = pltpu.async_copy(x_hbm.at[pl.ds(sid * stripe, stripe)],
                            xs.at[pl.ds(sid * stripe, stripe)], su1)
    cp_i.wait()
    cp_x.wait()
    plsc.subcore_barrier()

    rus = (ru0, ru1)
    rvs = (rv0, rv1)
    sus = (su0, su1)
    svs = (sv0, sv1)

    def fire(c, b):
        pltpu.async_copy(xs.at[idx2.at[c]], rus[b], sus[b])
        pltpu.async_copy(xs.at[idx2.at[c + _NCHUNK]], rvs[b], svs[b])

    def wait(b):
        pltpu.make_async_copy(xs.at[idx2.at[0]], rus[b], sus[b]).wait()
        pltpu.make_async_copy(xs.at[idx2.at[0]], rvs[b], svs[b]).wait()

    iota = lax.iota(jnp.int32, 16)

    def comp(c, b):
        rows_u, rows_v = rus[b], rvs[b]
        for g in range(_G):
            rows = g * 16 + iota

            def dstep(t, accs):
                a0, a1, a2, a3 = accs
                cols0 = iota + t * 4
                accs_new = [a0, a1, a2, a3]
                for k in range(4):
                    cols = (cols0 + k) & (_DP - 1)
                    u = plsc.bitcast(plsc.load_gather(rows_u, [rows, cols]),
                                     jnp.bfloat16)
                    v = plsc.bitcast(plsc.load_gather(rows_v, [rows, cols]),
                                     jnp.bfloat16)
                    lo, hi = plsc.unpack(u * v, format=plsc.PackFormat.INTERLEAVED)
                    accs_new[(2 * k) % 4] = accs_new[(2 * k) % 4] + lo
                    accs_new[(2 * k + 1) % 4] = accs_new[(2 * k + 1) % 4] + hi
                return tuple(accs_new)

            z = jnp.zeros((16,), jnp.float32)
            a0, a1, a2, a3 = lax.fori_loop(0, _DP // 4, dstep, (z, z, z, z))
            out_v[pl.ds(c * _C + g * 16, 16)] = (a0 + a1) + (a2 + a3)

    fire(0, 0)

    def loop_body(t, _):
        for b in range(2):
            c = 2 * t + b
            wait(b)
            fire(c + 1, 1 - b)
            comp(c, b)
        return 0

    lax.fori_loop(0, (_NCHUNK - 1) // 2, loop_body, 0)
    wait(0)
    comp(_NCHUNK - 1, 0)

    pltpu.sync_copy(out_v, out_hbm.at[pl.ds(wid * _EW, _EW)])


@functools.partial(jax.jit, static_argnums=())
def kernel(x, edge_index):
    src = edge_index[0].astype(jnp.int32).reshape(_NW, _NCHUNK, _C)
    dst = edge_index[1].astype(jnp.int32).reshape(_NW, _NCHUNK, _C)
    idx_cat = jnp.concatenate([src, dst], axis=1)
    x_packed = jax.lax.bitcast_convert_type(
        x.astype(jnp.bfloat16).reshape(_N_NODES, _DP, 2), jnp.int32)
    mesh = plsc.VectorSubcoreMesh(core_axis_name="c", subcore_axis_name="s")
    call = pl.kernel(
        _body,
        out_type=jax.ShapeDtypeStruct((_N_EDGES,), jnp.float32),
        mesh=mesh,
        scratch_types=[
            pltpu.VMEM((2 * _NCHUNK, _C), jnp.int32),
            pltpu.VMEM_SHARED((_N_NODES, _DP), jnp.int32),
            pltpu.VMEM((_C, _DP), jnp.int32),
            pltpu.VMEM((_C, _DP), jnp.int32),
            pltpu.VMEM((_C, _DP), jnp.int32),
            pltpu.VMEM((_C, _DP), jnp.int32),
            pltpu.VMEM((_EW,), jnp.float32),
            pltpu.SemaphoreType.DMA,
            pltpu.SemaphoreType.DMA,
            pltpu.SemaphoreType.DMA,
            pltpu.SemaphoreType.DMA,
        ],
        compiler_params=pltpu.CompilerParams(
            needs_layout_passes=False, use_tc_tiling_on_sc=False),
    )
    score = call(x_packed, idx_cat)
    return score.reshape(_N_EDGES, 1)


# speedup vs baseline: 1.2429x; 1.0821x over previous
import functools

import jax
import jax.numpy as jnp
from jax import lax
from jax.experimental import pallas as pl
from jax.experimental.pallas import tpu as pltpu
from jax.experimental.pallas import tpu_sc as plsc

_N_EDGES = 320000
_N_NODES = 10000
_D = 128
_DP = _D // 2
_NC = 2
_NS = 16
_NW = _NC * _NS
_EW = _N_EDGES // _NW
_C = 80
_NCHUNK = _EW // _C
_G = _C // 16


def _body(x_hbm, src_hbm, dst_hbm, out_hbm,
          idx_u, idx_v, xs, ru0, ru1, rv0, rv1, out_v,
          su0, su1, sv0, sv1):
    cid = lax.axis_index("c")
    sid = lax.axis_index("s")
    wid = sid * _NC + cid

    stripe = _N_NODES // _NS
    cp_u = pltpu.async_copy(src_hbm.at[wid], idx_u, su0)
    cp_v = pltpu.async_copy(dst_hbm.at[wid], idx_v, su1)
    cp_x = pltpu.async_copy(x_hbm.at[pl.ds(sid * stripe, stripe)],
                            xs.at[pl.ds(sid * stripe, stripe)], sv0)
    cp_u.wait()
    cp_v.wait()
    cp_x.wait()
    plsc.subcore_barrier()

    rus = (ru0, ru1)
    rvs = (rv0, rv1)
    sus = (su0, su1)
    svs = (sv0, sv1)

    def fire(c, b):
        pltpu.async_copy(xs.at[idx_u.at[c]], rus[b], sus[b])
        pltpu.async_copy(xs.at[idx_v.at[c]], rvs[b], svs[b])

    def wait(b):
        pltpu.make_async_copy(xs.at[idx_u.at[0]], rus[b], sus[b]).wait()
        pltpu.make_async_copy(xs.at[idx_v.at[0]], rvs[b], svs[b]).wait()

    iota = lax.iota(jnp.int32, 16)

    def comp(c, b):
        rows_u, rows_v = rus[b], rvs[b]
        for g in range(_G):
            rows = g * 16 + iota

            def dstep(t, accs):
                a0, a1, a2, a3 = accs
                cols0 = iota + t * 4
                accs_new = [a0, a1, a2, a3]
                for k in range(4):
                    cols = (cols0 + k) & (_DP - 1)
                    u = plsc.bitcast(plsc.load_gather(rows_u, [rows, cols]),
                                     jnp.bfloat16)
                    v = plsc.bitcast(plsc.load_gather(rows_v, [rows, cols]),
                                     jnp.bfloat16)
                    lo, hi = plsc.unpack(u * v, format=plsc.PackFormat.INTERLEAVED)
                    accs_new[(2 * k) % 4] = accs_new[(2 * k) % 4] + lo
                    accs_new[(2 * k + 1) % 4] = accs_new[(2 * k + 1) % 4] + hi
                return tuple(accs_new)

            z = jnp.zeros((16,), jnp.float32)
            a0, a1, a2, a3 = lax.fori_loop(0, _DP // 4, dstep, (z, z, z, z))
            out_v[pl.ds(c * _C + g * 16, 16)] = (a0 + a1) + (a2 + a3)

    fire(0, 0)

    def loop_body(t, _):
        for b in range(2):
            c = 2 * t + b
            wait(b)
            fire(c + 1, 1 - b)
            comp(c, b)
        return 0

    lax.fori_loop(0, (_NCHUNK - 1) // 2, loop_body, 0)
    wait(0)
    comp(_NCHUNK - 1, 0)

    pltpu.sync_copy(out_v, out_hbm.at[pl.ds(wid * _EW, _EW)])


@functools.partial(jax.jit, static_argnums=())
def kernel(x, edge_index):
    src = edge_index[0].astype(jnp.int32).reshape(_NW, _NCHUNK, _C)
    dst = edge_index[1].astype(jnp.int32).reshape(_NW, _NCHUNK, _C)
    x_packed = jax.lax.bitcast_convert_type(
        x.astype(jnp.bfloat16).reshape(_N_NODES, _DP, 2), jnp.int32)
    mesh = plsc.VectorSubcoreMesh(core_axis_name="c", subcore_axis_name="s")
    call = pl.kernel(
        _body,
        out_type=jax.ShapeDtypeStruct((_N_EDGES,), jnp.float32),
        mesh=mesh,
        scratch_types=[
            pltpu.VMEM((_NCHUNK, _C), jnp.int32),
            pltpu.VMEM((_NCHUNK, _C), jnp.int32),
            pltpu.VMEM_SHARED((_N_NODES, _DP), jnp.int32),
            pltpu.VMEM((_C, _DP), jnp.int32),
            pltpu.VMEM((_C, _DP), jnp.int32),
            pltpu.VMEM((_C, _DP), jnp.int32),
            pltpu.VMEM((_C, _DP), jnp.int32),
            pltpu.VMEM((_EW,), jnp.float32),
            pltpu.SemaphoreType.DMA,
            pltpu.SemaphoreType.DMA,
            pltpu.SemaphoreType.DMA,
            pltpu.SemaphoreType.DMA,
        ],
        compiler_params=pltpu.CompilerParams(
            needs_layout_passes=False, use_tc_tiling_on_sc=False),
    )
    score = call(x_packed, src, dst)
    return score.reshape(_N_EDGES, 1)
